# Initial kernel scaffold; baseline (speedup 1.0000x reference)
import functools

import jax
import jax.numpy as jnp
from jax import lax
from jax.experimental import pallas as pl
from jax.experimental.pallas import tpu as pltpu
from jax.experimental.pallas import tpu_sc as plsc

N = 10000
E = 320000
D = 128

NC = 2
NS = 16
NW = NC * NS
EPT = E // NW
CH = 125
NCH = EPT // CH
RPS = N // NS

_mesh = plsc.VectorSubcoreMesh(core_axis_name="c", subcore_axis_name="s")


@functools.partial(
    pl.kernel,
    out_type=jax.ShapeDtypeStruct((NW, N), jnp.float32),
    mesh=_mesh,
    scratch_types=[
        pltpu.VMEM((EPT,), jnp.int32),
        pltpu.VMEM((N,), jnp.float32),
    ],
)
def _degree_kernel(col_hbm, zeros_hbm, out_hbm, cols_v, hist_v):
    c = lax.axis_index("c")
    s = lax.axis_index("s")
    wid = c * NS + s
    pltpu.sync_copy(col_hbm.at[pl.ds(wid * EPT, EPT)], cols_v)
    pltpu.sync_copy(zeros_hbm, hist_v)
    ones = jnp.ones((16,), jnp.float32)

    @pl.loop(0, EPT // 16)
    def _(i):
        idx = cols_v[pl.ds(i * 16, 16)]
        plsc.addupdate_scatter(hist_v, [idx], ones)

    pltpu.sync_copy(hist_v, out_hbm.at[wid])


@functools.partial(
    pl.kernel,
    out_type=jax.ShapeDtypeStruct((NC, N, D), jnp.float32),
    mesh=_mesh,
    scratch_types=[
        pltpu.VMEM((NCH, CH), jnp.int32),
        pltpu.VMEM((NCH, CH), jnp.int32),
        pltpu.VMEM((CH, D), jnp.float32),
        pltpu.VMEM_SHARED((N, D), jnp.float32),
        pltpu.SemaphoreType.DMA,
    ],
)
def _aggregate_kernel(src_hbm, row_hbm, col_hbm, zeros_hbm, out_hbm,
                      rows_v, cols_v, buf, acc_sh, sem):
    c = lax.axis_index("c")
    s = lax.axis_index("s")
    wid = c * NS + s
    pltpu.sync_copy(row_hbm.at[wid], rows_v)
    pltpu.sync_copy(col_hbm.at[wid], cols_v)
    pltpu.sync_copy(zeros_hbm.at[pl.ds(s * RPS, RPS)],
                    acc_sh.at[pl.ds(s * RPS, RPS)])
    plsc.subcore_barrier()

    @pl.loop(0, NCH)
    def _(j):
        pltpu.async_copy(src_hbm.at[rows_v.at[j]], buf, sem).wait()
        pltpu.sync_copy(buf, acc_sh.at[cols_v.at[j]], add=True)

    plsc.subcore_barrier()
    pltpu.sync_copy(acc_sh.at[pl.ds(s * RPS, RPS)],
                    out_hbm.at[c, pl.ds(s * RPS, RPS)])


BLK = 1000


def _mm_body(x_ref, w_ref, o_ref):
    o_ref[...] = lax.dot_general(
        x_ref[...], w_ref[...], (((1,), (1,)), ((), ())),
        preferred_element_type=jnp.float32)


def _mm(x, w):
    return pl.pallas_call(
        _mm_body,
        grid=(N // BLK,),
        in_specs=[
            pl.BlockSpec((BLK, D), lambda i: (i, 0)),
            pl.BlockSpec((D, D), lambda i: (0, 0)),
        ],
        out_specs=pl.BlockSpec((BLK, D), lambda i: (i, 0)),
        out_shape=jax.ShapeDtypeStruct((N, D), jnp.float32),
    )(x, w)


def _norm_body(hist_ref, xw_ref, xwp_ref, dis_ref):
    deg = jnp.sum(hist_ref[...], axis=0) + 1.0
    dis = lax.rsqrt(deg)
    dis_ref[...] = dis[:, None]
    xwp_ref[...] = xw_ref[...] * dis[:, None]


def _norm(hist, xw):
    return pl.pallas_call(
        _norm_body,
        grid=(N // BLK,),
        in_specs=[
            pl.BlockSpec((NW, BLK), lambda i: (0, i)),
            pl.BlockSpec((BLK, D), lambda i: (i, 0)),
        ],
        out_specs=[
            pl.BlockSpec((BLK, D), lambda i: (i, 0)),
            pl.BlockSpec((BLK, 1), lambda i: (i, 0)),
        ],
        out_shape=[
            jax.ShapeDtypeStruct((N, D), jnp.float32),
            jax.ShapeDtypeStruct((N, 1), jnp.float32),
        ],
    )(hist, xw)


def _layer2_body(acc_ref, xwp_ref, dis_ref, b_ref, w_ref, o_ref):
    pre = acc_ref[0] + acc_ref[1] + xwp_ref[...]
    h = jnp.maximum(pre * dis_ref[...] + b_ref[...], 0.0)
    o_ref[...] = lax.dot_general(
        h, w_ref[...], (((1,), (1,)), ((), ())),
        preferred_element_type=jnp.float32) * dis_ref[...]


def _layer2(acc, xwp, dis, b, w):
    return pl.pallas_call(
        _layer2_body,
        grid=(N // BLK,),
        in_specs=[
            pl.BlockSpec((NC, BLK, D), lambda i: (0, i, 0)),
            pl.BlockSpec((BLK, D), lambda i: (i, 0)),
            pl.BlockSpec((BLK, 1), lambda i: (i, 0)),
            pl.BlockSpec((1, D), lambda i: (0, 0)),
            pl.BlockSpec((D, D), lambda i: (0, 0)),
        ],
        out_specs=pl.BlockSpec((BLK, D), lambda i: (i, 0)),
        out_shape=jax.ShapeDtypeStruct((N, D), jnp.float32),
    )(acc, xwp, dis, b, w)


def _combine_body(acc_ref, xwp_ref, dis_ref, b_ref, o_ref):
    pre = acc_ref[0] + acc_ref[1] + xwp_ref[...]
    o_ref[...] = pre * dis_ref[...] + b_ref[...]


def _combine(acc, xwp, dis, b):
    return pl.pallas_call(
        _combine_body,
        grid=(N // BLK,),
        in_specs=[
            pl.BlockSpec((NC, BLK, D), lambda i: (0, i, 0)),
            pl.BlockSpec((BLK, D), lambda i: (i, 0)),
            pl.BlockSpec((BLK, 1), lambda i: (i, 0)),
            pl.BlockSpec((1, D), lambda i: (0, 0)),
        ],
        out_specs=pl.BlockSpec((BLK, D), lambda i: (i, 0)),
        out_shape=jax.ShapeDtypeStruct((N, D), jnp.float32),
    )(acc, xwp, dis, b)


def kernel(x, edge_index, W1, b1, W2, b2):
    row = edge_index[0].reshape(NW, NCH, CH)
    col = edge_index[1].reshape(NW, NCH, CH)
    col_flat = edge_index[1]
    zeros_nd = jnp.zeros((N, D), jnp.float32)
    zeros_n = jnp.zeros((N,), jnp.float32)
    b1r = b1.reshape(1, D)
    b2r = b2.reshape(1, D)

    hist = _degree_kernel(col_flat, zeros_n)
    xw1 = _mm(x, W1)
    xw1p, dis = _norm(hist, xw1)
    acc1 = _aggregate_kernel(xw1p, row, col, zeros_nd)
    xw2p = _layer2(acc1, xw1p, dis, b1r, W2)
    acc2 = _aggregate_kernel(xw2p, row, col, zeros_nd)
    return _combine(acc2, xw2p, dis, b2r)


# speedup vs baseline: 18.3165x; 18.3165x over previous
import dataclasses
import functools

import jax
import jax.numpy as jnp
from jax import lax
from jax.experimental import pallas as pl
from jax.experimental.pallas import tpu as pltpu
from jax.experimental.pallas import tpu_sc as plsc

N = 10000
E = 320000
D = 128

NC = 2
NS = 16
NW = NC * NS
EPT = E // NW
CH = 125
NCH = EPT // CH
NP = 10112
RPS = NP // NS

_mesh = plsc.VectorSubcoreMesh(core_axis_name="c", subcore_axis_name="s")

_sc_params = pltpu.CompilerParams()
if "needs_layout_passes" in pltpu.CompilerParams.__dataclass_fields__:
    _sc_params = dataclasses.replace(_sc_params, needs_layout_passes=False)


@functools.partial(
    pl.kernel,
    out_type=jax.ShapeDtypeStruct((NW, N), jnp.float32),
    mesh=_mesh,
    scratch_types=[
        pltpu.VMEM((EPT,), jnp.int32),
        pltpu.VMEM((N,), jnp.float32),
    ],
    compiler_params=_sc_params,
)
def _degree_kernel(col_hbm, zeros_hbm, out_hbm, cols_v, hist_v):
    c = lax.axis_index("c")
    s = lax.axis_index("s")
    wid = c * NS + s
    pltpu.sync_copy(col_hbm.at[pl.ds(wid * EPT, EPT)], cols_v)
    pltpu.sync_copy(zeros_hbm, hist_v)
    ones = jnp.ones((16,), jnp.float32)

    @pl.loop(0, EPT // 16)
    def _(i):
        idx = cols_v[pl.ds(i * 16, 16)]
        plsc.addupdate_scatter(hist_v, [idx], ones)

    pltpu.sync_copy(hist_v, out_hbm.at[wid])


@functools.partial(
    pl.kernel,
    out_type=jax.ShapeDtypeStruct((NC, NP, D), jnp.float32),
    mesh=_mesh,
    scratch_types=[
        pltpu.VMEM((NCH, CH), jnp.int32),
        pltpu.VMEM((NCH, CH), jnp.int32),
        pltpu.VMEM((CH, D), jnp.float32),
        pltpu.VMEM_SHARED((NP, D), jnp.float32),
        pltpu.SemaphoreType.DMA,
    ],
)
def _aggregate_kernel(src_hbm, row_hbm, col_hbm, zeros_hbm, out_hbm,
                      rows_v, cols_v, buf, acc_sh, sem):
    c = lax.axis_index("c")
    s = lax.axis_index("s")
    wid = c * NS + s
    pltpu.sync_copy(row_hbm.at[wid], rows_v)
    pltpu.sync_copy(col_hbm.at[wid], cols_v)
    pltpu.sync_copy(zeros_hbm.at[pl.ds(s * RPS, RPS)],
                    acc_sh.at[pl.ds(s * RPS, RPS)])
    plsc.subcore_barrier()

    @pl.loop(0, NCH)
    def _(j):
        pltpu.async_copy(src_hbm.at[rows_v.at[j]], buf, sem).wait()
        pltpu.sync_copy(buf, acc_sh.at[cols_v.at[j]], add=True)

    plsc.subcore_barrier()
    pltpu.sync_copy(acc_sh.at[pl.ds(s * RPS, RPS)],
                    out_hbm.at[c, pl.ds(s * RPS, RPS)])


BLK = 1000


def _mm_body(x_ref, w_ref, o_ref):
    o_ref[...] = lax.dot_general(
        x_ref[...], w_ref[...], (((1,), (1,)), ((), ())),
        preferred_element_type=jnp.float32)


def _mm(x, w):
    return pl.pallas_call(
        _mm_body,
        grid=(N // BLK,),
        in_specs=[
            pl.BlockSpec((BLK, D), lambda i: (i, 0)),
            pl.BlockSpec((D, D), lambda i: (0, 0)),
        ],
        out_specs=pl.BlockSpec((BLK, D), lambda i: (i, 0)),
        out_shape=jax.ShapeDtypeStruct((N, D), jnp.float32),
    )(x, w)


def _norm_body(hist_ref, xw_ref, xwp_ref, dis_ref):
    deg = jnp.sum(hist_ref[...], axis=0) + 1.0
    dis = lax.rsqrt(deg)
    dis_ref[...] = dis
    xwp_ref[...] = xw_ref[...] * dis


def _norm(hist, xw):
    return pl.pallas_call(
        _norm_body,
        grid=(N // BLK,),
        in_specs=[
            pl.BlockSpec((NW, BLK, 1), lambda i: (0, i, 0)),
            pl.BlockSpec((BLK, D), lambda i: (i, 0)),
        ],
        out_specs=[
            pl.BlockSpec((BLK, D), lambda i: (i, 0)),
            pl.BlockSpec((BLK, 1), lambda i: (i, 0)),
        ],
        out_shape=[
            jax.ShapeDtypeStruct((N, D), jnp.float32),
            jax.ShapeDtypeStruct((N, 1), jnp.float32),
        ],
    )(hist, xw)


def _layer2_body(acc_ref, xwp_ref, dis_ref, b_ref, w_ref, o_ref):
    pre = acc_ref[0] + acc_ref[1] + xwp_ref[...]
    h = jnp.maximum(pre * dis_ref[...] + b_ref[...], 0.0)
    o_ref[...] = lax.dot_general(
        h, w_ref[...], (((1,), (1,)), ((), ())),
        preferred_element_type=jnp.float32) * dis_ref[...]


def _layer2(acc, xwp, dis, b, w):
    return pl.pallas_call(
        _layer2_body,
        grid=(N // BLK,),
        in_specs=[
            pl.BlockSpec((NC, BLK, D), lambda i: (0, i, 0)),
            pl.BlockSpec((BLK, D), lambda i: (i, 0)),
            pl.BlockSpec((BLK, 1), lambda i: (i, 0)),
            pl.BlockSpec((1, D), lambda i: (0, 0)),
            pl.BlockSpec((D, D), lambda i: (0, 0)),
        ],
        out_specs=pl.BlockSpec((BLK, D), lambda i: (i, 0)),
        out_shape=jax.ShapeDtypeStruct((N, D), jnp.float32),
    )(acc, xwp, dis, b, w)


def _combine_body(acc_ref, xwp_ref, dis_ref, b_ref, o_ref):
    pre = acc_ref[0] + acc_ref[1] + xwp_ref[...]
    o_ref[...] = pre * dis_ref[...] + b_ref[...]


def _combine(acc, xwp, dis, b):
    return pl.pallas_call(
        _combine_body,
        grid=(N // BLK,),
        in_specs=[
            pl.BlockSpec((NC, BLK, D), lambda i: (0, i, 0)),
            pl.BlockSpec((BLK, D), lambda i: (i, 0)),
            pl.BlockSpec((BLK, 1), lambda i: (i, 0)),
            pl.BlockSpec((1, D), lambda i: (0, 0)),
        ],
        out_specs=pl.BlockSpec((BLK, D), lambda i: (i, 0)),
        out_shape=jax.ShapeDtypeStruct((N, D), jnp.float32),
    )(acc, xwp, dis, b)


def kernel(x, edge_index, W1, b1, W2, b2):
    row = edge_index[0].reshape(NW, NCH, CH)
    col = edge_index[1].reshape(NW, NCH, CH)
    col_flat = edge_index[1]
    zeros_nd = jnp.zeros((NP, D), jnp.float32)
    zeros_n = jnp.zeros((N,), jnp.float32)
    b1r = b1.reshape(1, D)
    b2r = b2.reshape(1, D)

    hist = _degree_kernel(col_flat, zeros_n).reshape(NW, N, 1)
    xw1 = _mm(x, W1)
    xw1p, dis = _norm(hist, xw1)
    acc1 = _aggregate_kernel(xw1p, row, col, zeros_nd)
    xw2p = _layer2(acc1, xw1p, dis, b1r, W2)
    acc2 = _aggregate_kernel(xw2p, row, col, zeros_nd)
    return _combine(acc2, xw2p, dis, b2r)


# speedup vs baseline: 19.7105x; 1.0761x over previous
import dataclasses
import functools

import jax
import jax.numpy as jnp
from jax import lax
from jax.experimental import pallas as pl
from jax.experimental.pallas import tpu as pltpu
from jax.experimental.pallas import tpu_sc as plsc

N = 10000
E = 320000
D = 128

NC = 2
NS = 16
NW = NC * NS
EPT = E // NW
CH = 80
NCH = EPT // CH
NP = 10112
RPS = NP // NS

_mesh = plsc.VectorSubcoreMesh(core_axis_name="c", subcore_axis_name="s")

_sc_params = pltpu.CompilerParams()
if "needs_layout_passes" in pltpu.CompilerParams.__dataclass_fields__:
    _sc_params = dataclasses.replace(_sc_params, needs_layout_passes=False)


@functools.partial(
    pl.kernel,
    out_type=jax.ShapeDtypeStruct((NW, N), jnp.float32),
    mesh=_mesh,
    scratch_types=[
        pltpu.VMEM((EPT,), jnp.int32),
        pltpu.VMEM((N,), jnp.float32),
    ],
    compiler_params=_sc_params,
)
def _degree_kernel(col_hbm, zeros_hbm, out_hbm, cols_v, hist_v):
    c = lax.axis_index("c")
    s = lax.axis_index("s")
    wid = c * NS + s
    pltpu.sync_copy(col_hbm.at[pl.ds(wid * EPT, EPT)], cols_v)
    pltpu.sync_copy(zeros_hbm, hist_v)
    ones = jnp.ones((16,), jnp.float32)

    @pl.loop(0, EPT // 16)
    def _(i):
        idx = cols_v[pl.ds(i * 16, 16)]
        plsc.addupdate_scatter(hist_v, [idx], ones)

    pltpu.sync_copy(hist_v, out_hbm.at[wid])


@functools.partial(
    pl.kernel,
    out_type=jax.ShapeDtypeStruct((NC, NP, D), jnp.float32),
    mesh=_mesh,
    scratch_types=[
        pltpu.VMEM((EPT,), jnp.int32),
        pltpu.VMEM((NCH, CH), jnp.int32),
        pltpu.VMEM((2, CH, D), jnp.float32),
        pltpu.VMEM_SHARED((NP, D), jnp.float32),
        pltpu.SemaphoreType.DMA,
        pltpu.SemaphoreType.DMA,
        pltpu.SemaphoreType.DMA,
        pltpu.SemaphoreType.DMA,
    ],
)
def _aggregate_kernel(src_hbm, row_hbm, col_hbm, zeros_hbm, out_hbm,
                      rowsf_v, cols_v, dbuf, acc_sh, g0, g1, s0, s1):
    c = lax.axis_index("c")
    s = lax.axis_index("s")
    wid = c * NS + s
    bufs = (dbuf.at[0], dbuf.at[1])
    gsem = (g0, g1)
    ssem = (s0, s1)
    pltpu.sync_copy(row_hbm.at[wid], rowsf_v)
    pltpu.sync_copy(col_hbm.at[wid], cols_v)
    pltpu.sync_copy(zeros_hbm.at[pl.ds(s * RPS, RPS)],
                    acc_sh.at[pl.ds(s * RPS, RPS)])
    plsc.subcore_barrier()

    pltpu.async_copy(src_hbm.at[rowsf_v.at[pl.ds(0, CH)]], bufs[0], gsem[0])

    NMAIN = NCH - (NCH % 2)

    @pl.loop(0, NMAIN, step=2)
    def _(j):
        for b in range(2):
            k = j + b
            o = 1 - b
            pltpu.make_async_copy(src_hbm.at[rowsf_v.at[pl.ds(k * CH, CH)]],
                                  bufs[b], gsem[b]).wait()
            pltpu.async_copy(bufs[b], acc_sh.at[cols_v.at[k]],
                             ssem[b], add=True)

            @pl.when(k >= 1)
            def _():
                pltpu.make_async_copy(bufs[o], acc_sh.at[cols_v.at[k]],
                                      ssem[o]).wait()

            @pl.when(k + 1 < NCH)
            def _():
                pltpu.async_copy(src_hbm.at[rowsf_v.at[pl.ds((k + 1) * CH, CH)]],
                                 bufs[o], gsem[o])

    if NCH % 2:
        pltpu.make_async_copy(src_hbm.at[rowsf_v.at[pl.ds((NCH - 1) * CH, CH)]],
                              bufs[(NCH - 1) % 2], gsem[(NCH - 1) % 2]).wait()
        pltpu.async_copy(bufs[(NCH - 1) % 2], acc_sh.at[cols_v.at[NCH - 1]],
                         ssem[(NCH - 1) % 2], add=True)
        pltpu.make_async_copy(bufs[(NCH - 2) % 2], acc_sh.at[cols_v.at[0]],
                              ssem[(NCH - 2) % 2]).wait()
    pltpu.make_async_copy(bufs[(NCH - 1) % 2], acc_sh.at[cols_v.at[0]],
                          ssem[(NCH - 1) % 2]).wait()

    plsc.subcore_barrier()
    pltpu.sync_copy(acc_sh.at[pl.ds(s * RPS, RPS)],
                    out_hbm.at[c, pl.ds(s * RPS, RPS)])


BLK = 1000


def _mm_body(x_ref, w_ref, o_ref):
    o_ref[...] = lax.dot_general(
        x_ref[...], w_ref[...], (((1,), (1,)), ((), ())),
        preferred_element_type=jnp.float32)


def _mm(x, w):
    return pl.pallas_call(
        _mm_body,
        grid=(N // BLK,),
        in_specs=[
            pl.BlockSpec((BLK, D), lambda i: (i, 0)),
            pl.BlockSpec((D, D), lambda i: (0, 0)),
        ],
        out_specs=pl.BlockSpec((BLK, D), lambda i: (i, 0)),
        out_shape=jax.ShapeDtypeStruct((N, D), jnp.float32),
    )(x, w)


def _norm_body(hist_ref, xw_ref, xwp_ref, dis_ref):
    deg = jnp.sum(hist_ref[...], axis=0) + 1.0
    dis = lax.rsqrt(deg)
    dis_ref[...] = dis
    xwp_ref[...] = xw_ref[...] * dis


def _norm(hist, xw):
    return pl.pallas_call(
        _norm_body,
        grid=(N // BLK,),
        in_specs=[
            pl.BlockSpec((NW, BLK, 1), lambda i: (0, i, 0)),
            pl.BlockSpec((BLK, D), lambda i: (i, 0)),
        ],
        out_specs=[
            pl.BlockSpec((BLK, D), lambda i: (i, 0)),
            pl.BlockSpec((BLK, 1), lambda i: (i, 0)),
        ],
        out_shape=[
            jax.ShapeDtypeStruct((N, D), jnp.float32),
            jax.ShapeDtypeStruct((N, 1), jnp.float32),
        ],
    )(hist, xw)


def _layer2_body(acc_ref, xwp_ref, dis_ref, b_ref, w_ref, o_ref):
    pre = acc_ref[0] + acc_ref[1] + xwp_ref[...]
    h = jnp.maximum(pre * dis_ref[...] + b_ref[...], 0.0)
    o_ref[...] = lax.dot_general(
        h, w_ref[...], (((1,), (1,)), ((), ())),
        preferred_element_type=jnp.float32) * dis_ref[...]


def _layer2(acc, xwp, dis, b, w):
    return pl.pallas_call(
        _layer2_body,
        grid=(N // BLK,),
        in_specs=[
            pl.BlockSpec((NC, BLK, D), lambda i: (0, i, 0)),
            pl.BlockSpec((BLK, D), lambda i: (i, 0)),
            pl.BlockSpec((BLK, 1), lambda i: (i, 0)),
            pl.BlockSpec((1, D), lambda i: (0, 0)),
            pl.BlockSpec((D, D), lambda i: (0, 0)),
        ],
        out_specs=pl.BlockSpec((BLK, D), lambda i: (i, 0)),
        out_shape=jax.ShapeDtypeStruct((N, D), jnp.float32),
    )(acc, xwp, dis, b, w)


def _combine_body(acc_ref, xwp_ref, dis_ref, b_ref, o_ref):
    pre = acc_ref[0] + acc_ref[1] + xwp_ref[...]
    o_ref[...] = pre * dis_ref[...] + b_ref[...]


def _combine(acc, xwp, dis, b):
    return pl.pallas_call(
        _combine_body,
        grid=(N // BLK,),
        in_specs=[
            pl.BlockSpec((NC, BLK, D), lambda i: (0, i, 0)),
            pl.BlockSpec((BLK, D), lambda i: (i, 0)),
            pl.BlockSpec((BLK, 1), lambda i: (i, 0)),
            pl.BlockSpec((1, D), lambda i: (0, 0)),
        ],
        out_specs=pl.BlockSpec((BLK, D), lambda i: (i, 0)),
        out_shape=jax.ShapeDtypeStruct((N, D), jnp.float32),
    )(acc, xwp, dis, b)


def kernel(x, edge_index, W1, b1, W2, b2):
    row = edge_index[0].reshape(NW, EPT)
    col = edge_index[1].reshape(NW, NCH, CH)
    col_flat = edge_index[1]
    zeros_nd = jnp.zeros((NP, D), jnp.float32)
    zeros_n = jnp.zeros((N,), jnp.float32)
    b1r = b1.reshape(1, D)
    b2r = b2.reshape(1, D)

    hist = _degree_kernel(col_flat, zeros_n).reshape(NW, N, 1)
    xw1 = _mm(x, W1)
    xw1p, dis = _norm(hist, xw1)
    acc1 = _aggregate_kernel(xw1p, row, col, zeros_nd)
    xw2p = _layer2(acc1, xw1p, dis, b1r, W2)
    acc2 = _aggregate_kernel(xw2p, row, col, zeros_nd)
    return _combine(acc2, xw2p, dis, b2r)


# speedup vs baseline: 19.9180x; 1.0105x over previous
import dataclasses
import functools

import jax
import jax.numpy as jnp
from jax import lax
from jax.experimental import pallas as pl
from jax.experimental.pallas import tpu as pltpu
from jax.experimental.pallas import tpu_sc as plsc

N = 10000
E = 320000
D = 128

NC = 2
NS = 16
NW = NC * NS
EPT = E // NW
CH = 80
NCH = EPT // CH
NP = 10112
RPS = NP // NS

_mesh = plsc.VectorSubcoreMesh(core_axis_name="c", subcore_axis_name="s")

_sc_params = pltpu.CompilerParams()
if "needs_layout_passes" in pltpu.CompilerParams.__dataclass_fields__:
    _sc_params = dataclasses.replace(_sc_params, needs_layout_passes=False)


@functools.partial(
    pl.kernel,
    out_type=jax.ShapeDtypeStruct((NW, N), jnp.float32),
    mesh=_mesh,
    scratch_types=[
        pltpu.VMEM((EPT,), jnp.int32),
        pltpu.VMEM((N,), jnp.float32),
    ],
    compiler_params=_sc_params,
)
def _degree_kernel(col_hbm, zeros_hbm, out_hbm, cols_v, hist_v):
    c = lax.axis_index("c")
    s = lax.axis_index("s")
    wid = c * NS + s
    pltpu.sync_copy(col_hbm.at[pl.ds(wid * EPT, EPT)], cols_v)
    pltpu.sync_copy(zeros_hbm, hist_v)
    ones = jnp.ones((16,), jnp.float32)

    @pl.loop(0, EPT // 16)
    def _(i):
        idx = cols_v[pl.ds(i * 16, 16)]
        plsc.addupdate_scatter(hist_v, [idx], ones)

    pltpu.sync_copy(hist_v, out_hbm.at[wid])


@functools.partial(
    pl.kernel,
    out_type=jax.ShapeDtypeStruct((NC, NP, D), jnp.float32),
    mesh=_mesh,
    scratch_types=[
        pltpu.VMEM((EPT,), jnp.int32),
        pltpu.VMEM((NCH, CH), jnp.int32),
        pltpu.VMEM((2, CH, D), jnp.float32),
        pltpu.VMEM_SHARED((NP, D), jnp.float32),
        pltpu.SemaphoreType.DMA,
        pltpu.SemaphoreType.DMA,
        pltpu.SemaphoreType.DMA,
        pltpu.SemaphoreType.DMA,
    ],
)
def _aggregate_kernel(src_hbm, row_hbm, col_hbm, zeros_hbm, out_hbm,
                      rowsf_v, cols_v, dbuf, acc_sh, g0, g1, s0, s1):
    c = lax.axis_index("c")
    s = lax.axis_index("s")
    wid = c * NS + s
    bufs = (dbuf.at[0], dbuf.at[1])
    gsem = (g0, g1)
    ssem = (s0, s1)
    pltpu.sync_copy(row_hbm.at[wid], rowsf_v)
    pltpu.sync_copy(col_hbm.at[wid], cols_v)
    pltpu.sync_copy(zeros_hbm.at[pl.ds(s * RPS, RPS)],
                    acc_sh.at[pl.ds(s * RPS, RPS)])
    plsc.subcore_barrier()

    pltpu.async_copy(src_hbm.at[rowsf_v.at[pl.ds(0, CH)]], bufs[0], gsem[0])

    NMAIN = NCH - (NCH % 2)

    @pl.loop(0, NMAIN, step=2)
    def _(j):
        for b in range(2):
            k = j + b
            o = 1 - b
            pltpu.make_async_copy(src_hbm.at[rowsf_v.at[pl.ds(k * CH, CH)]],
                                  bufs[b], gsem[b]).wait()
            pltpu.async_copy(bufs[b], acc_sh.at[cols_v.at[k]],
                             ssem[b], add=True)

            @pl.when(k >= 1)
            def _():
                pltpu.make_async_copy(bufs[o], acc_sh.at[cols_v.at[k]],
                                      ssem[o]).wait()

            @pl.when(k + 1 < NCH)
            def _():
                pltpu.async_copy(src_hbm.at[rowsf_v.at[pl.ds((k + 1) * CH, CH)]],
                                 bufs[o], gsem[o])

    if NCH % 2:
        pltpu.make_async_copy(src_hbm.at[rowsf_v.at[pl.ds((NCH - 1) * CH, CH)]],
                              bufs[(NCH - 1) % 2], gsem[(NCH - 1) % 2]).wait()
        pltpu.async_copy(bufs[(NCH - 1) % 2], acc_sh.at[cols_v.at[NCH - 1]],
                         ssem[(NCH - 1) % 2], add=True)
        pltpu.make_async_copy(bufs[(NCH - 2) % 2], acc_sh.at[cols_v.at[0]],
                              ssem[(NCH - 2) % 2]).wait()
    pltpu.make_async_copy(bufs[(NCH - 1) % 2], acc_sh.at[cols_v.at[0]],
                          ssem[(NCH - 1) % 2]).wait()

    plsc.subcore_barrier()
    pltpu.sync_copy(acc_sh.at[pl.ds(s * RPS, RPS)],
                    out_hbm.at[c, pl.ds(s * RPS, RPS)])


BLK = 1000


def _norm_body(hist_ref, x_ref, w_ref, xwp_ref, dis_ref):
    deg = jnp.sum(hist_ref[...], axis=0) + 1.0
    dis = lax.rsqrt(deg)
    dis_ref[...] = dis
    xwp_ref[...] = lax.dot_general(
        x_ref[...] * dis, w_ref[...], (((1,), (1,)), ((), ())),
        preferred_element_type=jnp.float32)


def _norm(hist, x, w):
    return pl.pallas_call(
        _norm_body,
        grid=(N // BLK,),
        in_specs=[
            pl.BlockSpec((NW, BLK, 1), lambda i: (0, i, 0)),
            pl.BlockSpec((BLK, D), lambda i: (i, 0)),
            pl.BlockSpec((D, D), lambda i: (0, 0)),
        ],
        out_specs=[
            pl.BlockSpec((BLK, D), lambda i: (i, 0)),
            pl.BlockSpec((BLK, 1), lambda i: (i, 0)),
        ],
        out_shape=[
            jax.ShapeDtypeStruct((N, D), jnp.float32),
            jax.ShapeDtypeStruct((N, 1), jnp.float32),
        ],
    )(hist, x, w)


def _layer2_body(acc_ref, xwp_ref, dis_ref, b_ref, w_ref, o_ref):
    pre = acc_ref[0] + acc_ref[1] + xwp_ref[...]
    h = jnp.maximum(pre * dis_ref[...] + b_ref[...], 0.0)
    o_ref[...] = lax.dot_general(
        h, w_ref[...], (((1,), (1,)), ((), ())),
        preferred_element_type=jnp.float32) * dis_ref[...]


def _layer2(acc, xwp, dis, b, w):
    return pl.pallas_call(
        _layer2_body,
        grid=(N // BLK,),
        in_specs=[
            pl.BlockSpec((NC, BLK, D), lambda i: (0, i, 0)),
            pl.BlockSpec((BLK, D), lambda i: (i, 0)),
            pl.BlockSpec((BLK, 1), lambda i: (i, 0)),
            pl.BlockSpec((1, D), lambda i: (0, 0)),
            pl.BlockSpec((D, D), lambda i: (0, 0)),
        ],
        out_specs=pl.BlockSpec((BLK, D), lambda i: (i, 0)),
        out_shape=jax.ShapeDtypeStruct((N, D), jnp.float32),
    )(acc, xwp, dis, b, w)


def _combine_body(acc_ref, xwp_ref, dis_ref, b_ref, o_ref):
    pre = acc_ref[0] + acc_ref[1] + xwp_ref[...]
    o_ref[...] = pre * dis_ref[...] + b_ref[...]


def _combine(acc, xwp, dis, b):
    return pl.pallas_call(
        _combine_body,
        grid=(N // BLK,),
        in_specs=[
            pl.BlockSpec((NC, BLK, D), lambda i: (0, i, 0)),
            pl.BlockSpec((BLK, D), lambda i: (i, 0)),
            pl.BlockSpec((BLK, 1), lambda i: (i, 0)),
            pl.BlockSpec((1, D), lambda i: (0, 0)),
        ],
        out_specs=pl.BlockSpec((BLK, D), lambda i: (i, 0)),
        out_shape=jax.ShapeDtypeStruct((N, D), jnp.float32),
    )(acc, xwp, dis, b)


def kernel(x, edge_index, W1, b1, W2, b2):
    row = edge_index[0].reshape(NW, EPT)
    col = edge_index[1].reshape(NW, NCH, CH)
    col_flat = edge_index[1]
    zeros_nd = jnp.zeros((NP, D), jnp.float32)
    zeros_n = jnp.zeros((N,), jnp.float32)
    b1r = b1.reshape(1, D)
    b2r = b2.reshape(1, D)

    hist = _degree_kernel(col_flat, zeros_n).reshape(NW, N, 1)
    xw1p, dis = _norm(hist, x, W1)
    acc1 = _aggregate_kernel(xw1p, row, col, zeros_nd)
    xw2p = _layer2(acc1, xw1p, dis, b1r, W2)
    acc2 = _aggregate_kernel(xw2p, row, col, zeros_nd)
    return _combine(acc2, xw2p, dis, b2r)


# speedup vs baseline: 23.4090x; 1.1753x over previous
import dataclasses
import functools

import jax
import jax.numpy as jnp
from jax import lax
from jax.experimental import pallas as pl
from jax.experimental.pallas import tpu as pltpu
from jax.experimental.pallas import tpu_sc as plsc

N = 10000
E = 320000
D = 128

NC = 2
NS = 16
NW = NC * NS
EPT = E // NW
CH = 80
NCH = EPT // CH
NP = 10112
RPS = NP // NS

_mesh = plsc.VectorSubcoreMesh(core_axis_name="c", subcore_axis_name="s")

_sc_params = pltpu.CompilerParams()
if "needs_layout_passes" in pltpu.CompilerParams.__dataclass_fields__:
    _sc_params = dataclasses.replace(_sc_params, needs_layout_passes=False)


@functools.partial(
    pl.kernel,
    out_type=jax.ShapeDtypeStruct((NW, N), jnp.float32),
    mesh=_mesh,
    scratch_types=[
        pltpu.VMEM((EPT,), jnp.int32),
        pltpu.VMEM((N,), jnp.float32),
    ],
    compiler_params=_sc_params,
)
def _degree_kernel(col_hbm, zeros_hbm, out_hbm, cols_v, hist_v):
    c = lax.axis_index("c")
    s = lax.axis_index("s")
    wid = c * NS + s
    pltpu.sync_copy(col_hbm.at[pl.ds(wid * EPT, EPT)], cols_v)
    pltpu.sync_copy(zeros_hbm, hist_v)
    ones = jnp.ones((16,), jnp.float32)

    @pl.loop(0, EPT // 16)
    def _(i):
        idx = cols_v[pl.ds(i * 16, 16)]
        plsc.addupdate_scatter(hist_v, [idx], ones)

    pltpu.sync_copy(hist_v, out_hbm.at[wid])


NBUF = 4
NST = 8
NTAIL = 5

assert NCH % NST == NTAIL


@functools.partial(
    pl.kernel,
    out_type=jax.ShapeDtypeStruct((NC, NP, D), jnp.float32),
    mesh=_mesh,
    scratch_types=[
        pltpu.VMEM((NST, 2, CH), jnp.int32),
        pltpu.VMEM((NBUF, CH, D), jnp.float32),
        pltpu.VMEM_SHARED((NP, D), jnp.float32),
    ] + [pltpu.SemaphoreType.DMA] * (2 * NBUF + NST),
)
def _aggregate_kernel(src_hbm, idx_hbm, zeros_hbm, out_hbm,
                      stage, dbuf, acc_sh, *sems):
    gsem = sems[:NBUF]
    ssem = sems[NBUF:2 * NBUF]
    isem = sems[2 * NBUF:]
    c = lax.axis_index("c")
    s = lax.axis_index("s")
    wid = c * NS + s
    pltpu.sync_copy(zeros_hbm.at[pl.ds(s * RPS, RPS)],
                    acc_sh.at[pl.ds(s * RPS, RPS)])
    for k in range(NBUF):
        pltpu.async_copy(idx_hbm.at[wid, k], stage.at[k], isem[k])
    for k in range(2):
        pltpu.make_async_copy(idx_hbm.at[wid, k], stage.at[k], isem[k]).wait()
        pltpu.async_copy(src_hbm.at[stage.at[k, 0]], dbuf.at[k], gsem[k])
    plsc.subcore_barrier()

    def _slot(k, b, traced):
        bb = b % NBUF
        b2 = (b + 2) % NBUF
        st2 = (b + 2) % NST
        st4 = (b + 4) % NST
        pltpu.make_async_copy(src_hbm.at[stage.at[b, 0]],
                              dbuf.at[bb], gsem[bb]).wait()
        pltpu.async_copy(dbuf.at[bb], acc_sh.at[stage.at[b, 1]],
                         ssem[bb], add=True)

        def _drain():
            pltpu.make_async_copy(dbuf.at[b2], acc_sh.at[stage.at[b, 1]],
                                  ssem[b2]).wait()

        def _next_gather():
            pltpu.make_async_copy(idx_hbm.at[wid, k + 2], stage.at[st2],
                                  isem[st2]).wait()
            pltpu.async_copy(src_hbm.at[stage.at[st2, 0]],
                             dbuf.at[b2], gsem[b2])

        def _next_idx():
            pltpu.async_copy(idx_hbm.at[wid, k + 4], stage.at[st4],
                             isem[st4])

        if traced:
            pl.when(k >= 2)(_drain)
            _next_gather()
            _next_idx()
        else:
            _drain()
            if k + 2 < NCH:
                _next_gather()
            if k + 4 < NCH:
                _next_idx()

    @pl.loop(0, NCH - NTAIL, step=NST)
    def _(j):
        for b in range(NST):
            _slot(j + b, b, True)

    for k in range(NCH - NTAIL, NCH):
        _slot(k, k % NST, False)

    pltpu.make_async_copy(dbuf.at[(NCH - 2) % NBUF],
                          acc_sh.at[stage.at[(NCH - 2) % NST, 1]],
                          ssem[(NCH - 2) % NBUF]).wait()
    pltpu.make_async_copy(dbuf.at[(NCH - 1) % NBUF],
                          acc_sh.at[stage.at[(NCH - 1) % NST, 1]],
                          ssem[(NCH - 1) % NBUF]).wait()

    plsc.subcore_barrier()
    pltpu.sync_copy(acc_sh.at[pl.ds(s * RPS, RPS)],
                    out_hbm.at[c, pl.ds(s * RPS, RPS)])


BLK = 1000


def _norm_body(hist_ref, x_ref, w_ref, xwp_ref, dis_ref):
    deg = jnp.sum(hist_ref[...], axis=0) + 1.0
    dis = lax.rsqrt(deg)
    dis_ref[...] = dis
    xwp_ref[...] = lax.dot_general(
        x_ref[...] * dis, w_ref[...], (((1,), (1,)), ((), ())),
        preferred_element_type=jnp.float32)


def _norm(hist, x, w):
    return pl.pallas_call(
        _norm_body,
        grid=(N // BLK,),
        in_specs=[
            pl.BlockSpec((NW, BLK, 1), lambda i: (0, i, 0)),
            pl.BlockSpec((BLK, D), lambda i: (i, 0)),
            pl.BlockSpec((D, D), lambda i: (0, 0)),
        ],
        out_specs=[
            pl.BlockSpec((BLK, D), lambda i: (i, 0)),
            pl.BlockSpec((BLK, 1), lambda i: (i, 0)),
        ],
        out_shape=[
            jax.ShapeDtypeStruct((N, D), jnp.float32),
            jax.ShapeDtypeStruct((N, 1), jnp.float32),
        ],
    )(hist, x, w)


def _layer2_body(acc_ref, xwp_ref, dis_ref, b_ref, w_ref, o_ref):
    pre = acc_ref[0] + acc_ref[1] + xwp_ref[...]
    h = jnp.maximum(pre * dis_ref[...] + b_ref[...], 0.0)
    o_ref[...] = lax.dot_general(
        h, w_ref[...], (((1,), (1,)), ((), ())),
        preferred_element_type=jnp.float32) * dis_ref[...]


def _layer2(acc, xwp, dis, b, w):
    return pl.pallas_call(
        _layer2_body,
        grid=(N // BLK,),
        in_specs=[
            pl.BlockSpec((NC, BLK, D), lambda i: (0, i, 0)),
            pl.BlockSpec((BLK, D), lambda i: (i, 0)),
            pl.BlockSpec((BLK, 1), lambda i: (i, 0)),
            pl.BlockSpec((1, D), lambda i: (0, 0)),
            pl.BlockSpec((D, D), lambda i: (0, 0)),
        ],
        out_specs=pl.BlockSpec((BLK, D), lambda i: (i, 0)),
        out_shape=jax.ShapeDtypeStruct((N, D), jnp.float32),
    )(acc, xwp, dis, b, w)


def _combine_body(acc_ref, xwp_ref, dis_ref, b_ref, o_ref):
    pre = acc_ref[0] + acc_ref[1] + xwp_ref[...]
    o_ref[...] = pre * dis_ref[...] + b_ref[...]


def _combine(acc, xwp, dis, b):
    return pl.pallas_call(
        _combine_body,
        grid=(N // BLK,),
        in_specs=[
            pl.BlockSpec((NC, BLK, D), lambda i: (0, i, 0)),
            pl.BlockSpec((BLK, D), lambda i: (i, 0)),
            pl.BlockSpec((BLK, 1), lambda i: (i, 0)),
            pl.BlockSpec((1, D), lambda i: (0, 0)),
        ],
        out_specs=pl.BlockSpec((BLK, D), lambda i: (i, 0)),
        out_shape=jax.ShapeDtypeStruct((N, D), jnp.float32),
    )(acc, xwp, dis, b)


def kernel(x, edge_index, W1, b1, W2, b2):
    row = edge_index[0].reshape(NW, NCH, 1, CH)
    col = edge_index[1].reshape(NW, NCH, 1, CH)
    idxc = jnp.concatenate([row, col], axis=2)
    col_flat = edge_index[1]
    zeros_nd = jnp.zeros((NP, D), jnp.float32)
    zeros_n = jnp.zeros((N,), jnp.float32)
    b1r = b1.reshape(1, D)
    b2r = b2.reshape(1, D)

    hist = _degree_kernel(col_flat, zeros_n).reshape(NW, N, 1)
    xw1p, dis = _norm(hist, x, W1)
    acc1 = _aggregate_kernel(xw1p, idxc, zeros_nd)
    xw2p = _layer2(acc1, xw1p, dis, b1r, W2)
    acc2 = _aggregate_kernel(xw2p, idxc, zeros_nd)
    return _combine(acc2, xw2p, dis, b2r)


# speedup vs baseline: 25.0865x; 1.0717x over previous
import dataclasses
import functools

import jax
import jax.numpy as jnp
from jax import lax
from jax.experimental import pallas as pl
from jax.experimental.pallas import tpu as pltpu
from jax.experimental.pallas import tpu_sc as plsc

N = 10000
E = 320000
D = 128

NC = 2
NS = 16
NW = NC * NS
EPT = E // NW
CH = 80
NCH = EPT // CH
NP = 10112
RPS = NP // NS

_mesh = plsc.VectorSubcoreMesh(core_axis_name="c", subcore_axis_name="s")

_sc_params = pltpu.CompilerParams()
if "needs_layout_passes" in pltpu.CompilerParams.__dataclass_fields__:
    _sc_params = dataclasses.replace(_sc_params, needs_layout_passes=False)


@functools.partial(
    pl.kernel,
    out_type=jax.ShapeDtypeStruct((NW, N), jnp.float32),
    mesh=_mesh,
    scratch_types=[
        pltpu.VMEM((EPT,), jnp.int32),
        pltpu.VMEM((N,), jnp.float32),
    ],
    compiler_params=_sc_params,
)
def _degree_kernel(col_hbm, zeros_hbm, out_hbm, cols_v, hist_v):
    c = lax.axis_index("c")
    s = lax.axis_index("s")
    wid = c * NS + s
    pltpu.sync_copy(col_hbm.at[pl.ds(wid * EPT, EPT)], cols_v)
    pltpu.sync_copy(zeros_hbm, hist_v)
    ones = jnp.ones((16,), jnp.float32)

    @pl.loop(0, EPT // 16)
    def _(i):
        idx = cols_v[pl.ds(i * 16, 16)]
        plsc.addupdate_scatter(hist_v, [idx], ones)

    pltpu.sync_copy(hist_v, out_hbm.at[wid])


NBUF = 4
NST = 8
NTAIL = 5

assert NCH % NST == NTAIL


@functools.partial(
    pl.kernel,
    out_type=jax.ShapeDtypeStruct((NC, NP, D), jnp.float32),
    mesh=_mesh,
    scratch_types=[
        pltpu.VMEM((NST, 2, CH), jnp.int32),
        pltpu.VMEM((NBUF, CH, D), jnp.float32),
        pltpu.VMEM_SHARED((NP, D), jnp.float32),
    ] + [pltpu.SemaphoreType.DMA] * (2 * NBUF + NST),
)
def _aggregate_kernel(src_hbm, idx_hbm, zeros_hbm, out_hbm,
                      stage, dbuf, acc_sh, *sems):
    gsem = sems[:NBUF]
    ssem = sems[NBUF:2 * NBUF]
    isem = sems[2 * NBUF:]
    c = lax.axis_index("c")
    s = lax.axis_index("s")
    wid = c * NS + s
    pltpu.sync_copy(zeros_hbm.at[pl.ds(s * RPS, RPS)],
                    acc_sh.at[pl.ds(s * RPS, RPS)])
    for k in range(5):
        pltpu.async_copy(idx_hbm.at[wid, k], stage.at[k], isem[k])
    for k in range(3):
        pltpu.make_async_copy(idx_hbm.at[wid, k], stage.at[k], isem[k]).wait()
        pltpu.async_copy(src_hbm.at[stage.at[k, 0]], dbuf.at[k], gsem[k])
    plsc.subcore_barrier()

    def _slot(k, b, traced):
        bb = b % NBUF
        b3 = (b + 3) % NBUF
        st3 = (b + 3) % NST
        st5 = (b + 5) % NST
        pltpu.make_async_copy(src_hbm.at[stage.at[b, 0]],
                              dbuf.at[bb], gsem[bb]).wait()
        pltpu.async_copy(dbuf.at[bb], acc_sh.at[stage.at[b, 1]],
                         ssem[bb], add=True)

        def _drain():
            pltpu.make_async_copy(dbuf.at[b3], acc_sh.at[stage.at[b, 1]],
                                  ssem[b3]).wait()

        def _next_gather():
            pltpu.make_async_copy(idx_hbm.at[wid, k + 3], stage.at[st3],
                                  isem[st3]).wait()
            pltpu.async_copy(src_hbm.at[stage.at[st3, 0]],
                             dbuf.at[b3], gsem[b3])

        def _next_idx():
            pltpu.async_copy(idx_hbm.at[wid, k + 5], stage.at[st5],
                             isem[st5])

        if traced:
            pl.when(k >= 1)(_drain)
            _next_gather()
            _next_idx()
        else:
            _drain()
            if k + 3 < NCH:
                _next_gather()
            if k + 5 < NCH:
                _next_idx()

    @pl.loop(0, NCH - NTAIL, step=NST)
    def _(j):
        for b in range(NST):
            _slot(j + b, b, True)

    for k in range(NCH - NTAIL, NCH):
        _slot(k, k % NST, False)

    pltpu.make_async_copy(dbuf.at[(NCH - 1) % NBUF],
                          acc_sh.at[stage.at[(NCH - 1) % NST, 1]],
                          ssem[(NCH - 1) % NBUF]).wait()

    plsc.subcore_barrier()
    pltpu.sync_copy(acc_sh.at[pl.ds(s * RPS, RPS)],
                    out_hbm.at[c, pl.ds(s * RPS, RPS)])


BLK = 1000


def _norm_body(hist_ref, x_ref, w_ref, xwp_ref, dis_ref):
    deg = jnp.sum(hist_ref[...], axis=0) + 1.0
    dis = lax.rsqrt(deg)
    dis_ref[...] = dis
    xwp_ref[...] = lax.dot_general(
        x_ref[...] * dis, w_ref[...], (((1,), (1,)), ((), ())),
        preferred_element_type=jnp.float32)


def _norm(hist, x, w):
    return pl.pallas_call(
        _norm_body,
        grid=(N // BLK,),
        in_specs=[
            pl.BlockSpec((NW, BLK, 1), lambda i: (0, i, 0)),
            pl.BlockSpec((BLK, D), lambda i: (i, 0)),
            pl.BlockSpec((D, D), lambda i: (0, 0)),
        ],
        out_specs=[
            pl.BlockSpec((BLK, D), lambda i: (i, 0)),
            pl.BlockSpec((BLK, 1), lambda i: (i, 0)),
        ],
        out_shape=[
            jax.ShapeDtypeStruct((N, D), jnp.float32),
            jax.ShapeDtypeStruct((N, 1), jnp.float32),
        ],
    )(hist, x, w)


def _layer2_body(acc_ref, xwp_ref, dis_ref, b_ref, w_ref, o_ref):
    pre = acc_ref[0] + acc_ref[1] + xwp_ref[...]
    h = jnp.maximum(pre * dis_ref[...] + b_ref[...], 0.0)
    o_ref[...] = lax.dot_general(
        h, w_ref[...], (((1,), (1,)), ((), ())),
        preferred_element_type=jnp.float32) * dis_ref[...]


def _layer2(acc, xwp, dis, b, w):
    return pl.pallas_call(
        _layer2_body,
        grid=(N // BLK,),
        in_specs=[
            pl.BlockSpec((NC, BLK, D), lambda i: (0, i, 0)),
            pl.BlockSpec((BLK, D), lambda i: (i, 0)),
            pl.BlockSpec((BLK, 1), lambda i: (i, 0)),
            pl.BlockSpec((1, D), lambda i: (0, 0)),
            pl.BlockSpec((D, D), lambda i: (0, 0)),
        ],
        out_specs=pl.BlockSpec((BLK, D), lambda i: (i, 0)),
        out_shape=jax.ShapeDtypeStruct((N, D), jnp.float32),
    )(acc, xwp, dis, b, w)


def _combine_body(acc_ref, xwp_ref, dis_ref, b_ref, o_ref):
    pre = acc_ref[0] + acc_ref[1] + xwp_ref[...]
    o_ref[...] = pre * dis_ref[...] + b_ref[...]


def _combine(acc, xwp, dis, b):
    return pl.pallas_call(
        _combine_body,
        grid=(N // BLK,),
        in_specs=[
            pl.BlockSpec((NC, BLK, D), lambda i: (0, i, 0)),
            pl.BlockSpec((BLK, D), lambda i: (i, 0)),
            pl.BlockSpec((BLK, 1), lambda i: (i, 0)),
            pl.BlockSpec((1, D), lambda i: (0, 0)),
        ],
        out_specs=pl.BlockSpec((BLK, D), lambda i: (i, 0)),
        out_shape=jax.ShapeDtypeStruct((N, D), jnp.float32),
    )(acc, xwp, dis, b)


def kernel(x, edge_index, W1, b1, W2, b2):
    row = edge_index[0].reshape(NW, NCH, 1, CH)
    col = edge_index[1].reshape(NW, NCH, 1, CH)
    idxc = jnp.concatenate([row, col], axis=2)
    col_flat = edge_index[1]
    zeros_nd = jnp.zeros((NP, D), jnp.float32)
    zeros_n = jnp.zeros((N,), jnp.float32)
    b1r = b1.reshape(1, D)
    b2r = b2.reshape(1, D)

    hist = _degree_kernel(col_flat, zeros_n).reshape(NW, N, 1)
    xw1p, dis = _norm(hist, x, W1)
    acc1 = _aggregate_kernel(xw1p, idxc, zeros_nd)
    xw2p = _layer2(acc1, xw1p, dis, b1r, W2)
    acc2 = _aggregate_kernel(xw2p, idxc, zeros_nd)
    return _combine(acc2, xw2p, dis, b2r)
